# baseline (speedup 1.0000x reference)
import functools

import jax
import jax.numpy as jnp
from jax import lax
from jax.experimental import pallas as pl
from jax.experimental.pallas import tpu as pltpu
from jax.experimental.pallas import tpu_sc as plsc

NC = 2
NS = 16
NW = NC * NS
LANES = 16
CHUNK = 128


def _sc_mesh():
    return plsc.VectorSubcoreMesh(
        core_axis_name="c", subcore_axis_name="s", num_cores=NC, num_subcores=NS
    )


def _wid(cid, sid):
    return sid * NC + cid


def _degree_call(src, dst, n, e):
    n_pad = _round_up(n + LANES, NS * CHUNK)
    rows_per_tile = n_pad // NS
    nchunk = e // CHUNK
    base_blk = nchunk // NW
    rem = nchunk % NW
    max_blk = base_blk + (1 if rem else 0)

    @functools.partial(
        pl.kernel,
        out_type=[
            jax.ShapeDtypeStruct((NC, n_pad, LANES), jnp.float32),
            jax.ShapeDtypeStruct((e,), jnp.int32),
        ],
        mesh=_sc_mesh(),
        scratch_types=[
            pltpu.VMEM((max_blk * CHUNK,), jnp.int32),
            pltpu.VMEM((max_blk * CHUNK,), jnp.int32),
            pltpu.VMEM((max_blk, CHUNK), jnp.int32),
            pltpu.VMEM((max_blk * CHUNK,), jnp.int32),
            pltpu.VMEM((CHUNK, LANES), jnp.float32),
            pltpu.VMEM((rows_per_tile, LANES), jnp.float32),
            pltpu.VMEM_SHARED((n_pad, LANES), jnp.float32),
            pltpu.SemaphoreType.DMA,
        ],
        compiler_params=pltpu.CompilerParams(use_tc_tiling_on_sc=False),
    )
    def deg_kernel(src_hbm, dst_hbm, deg_out, dstm_out,
                   srcs, dsts, srcm2, dstm_v, ones_v, zero_v, acc_sh, sem):
        cid = lax.axis_index("c")
        sid = lax.axis_index("s")
        wid = _wid(cid, sid)
        nblk = jnp.where(wid < rem, base_blk + 1, base_blk)
        cb = base_blk * wid + jnp.minimum(wid, rem)
        ebase = cb * CHUNK

        pltpu.sync_copy(src_hbm.at[pl.ds(ebase, base_blk * CHUNK)],
                        srcs.at[pl.ds(0, base_blk * CHUNK)])
        pltpu.sync_copy(dst_hbm.at[pl.ds(ebase, base_blk * CHUNK)],
                        dsts.at[pl.ds(0, base_blk * CHUNK)])
        if rem:
            @pl.when(wid < rem)
            def _tail():
                off = base_blk * CHUNK
                pltpu.sync_copy(src_hbm.at[pl.ds(ebase + off, CHUNK)],
                                srcs.at[pl.ds(off, CHUNK)])
                pltpu.sync_copy(dst_hbm.at[pl.ds(ebase + off, CHUNK)],
                                dsts.at[pl.ds(off, CHUNK)])

        zeros16 = jnp.zeros((LANES,), jnp.float32)
        e0 = jnp.where(lax.iota(jnp.int32, LANES) == 0, 1.0, 0.0)

        @pl.loop(0, CHUNK)
        def _fill(i):
            ones_v[i, :] = e0

        @pl.loop(0, rows_per_tile)
        def _fillz(i):
            zero_v[i, :] = zeros16

        pltpu.sync_copy(zero_v, acc_sh.at[pl.ds(sid * rows_per_tile,
                                                rows_per_tile)])

        dump = n + lax.iota(jnp.int32, LANES)

        @pl.loop(0, nblk)
        def _mask(r):
            for k in range(CHUNK // LANES):
                sl1 = pl.ds(r * CHUNK + k * LANES, LANES)
                s = srcs[sl1]
                d = dsts[sl1]
                is_loop = s == d
                srcm2[r, pl.ds(k * LANES, LANES)] = jnp.where(is_loop, dump, s)
                dstm_v[sl1] = jnp.where(is_loop, dump, d)

        pltpu.sync_copy(dstm_v.at[pl.ds(0, base_blk * CHUNK)],
                        dstm_out.at[pl.ds(ebase, base_blk * CHUNK)])
        if rem:
            @pl.when(wid < rem)
            def _tail2():
                off = base_blk * CHUNK
                pltpu.sync_copy(dstm_v.at[pl.ds(off, CHUNK)],
                                dstm_out.at[pl.ds(ebase + off, CHUNK)])

        plsc.subcore_barrier()

        @pl.loop(0, nblk)
        def _scat(r):
            pltpu.async_copy(ones_v, acc_sh.at[srcm2.at[r]], sem, add=True)

        @pl.loop(0, nblk)
        def _drain(r):
            pltpu.make_async_copy(
                deg_out.at[0, pl.ds(0, CHUNK)], ones_v, sem
            ).wait()

        plsc.subcore_barrier()
        off = pl.multiple_of(sid * rows_per_tile, 8)
        pltpu.sync_copy(
            acc_sh.at[pl.ds(off, rows_per_tile)],
            deg_out.at[cid, pl.ds(off, rows_per_tile)],
        )

    return deg_kernel(src, dst)


def _propagate_call(u, src, dstm2, n, e, ch):
    n_pad = _round_up(n + LANES, NS * CHUNK)
    rows_per_tile = n_pad // NS
    zchunks = rows_per_tile // CHUNK
    nchunk = e // CHUNK
    base_blk = nchunk // NW
    rem = nchunk % NW
    max_blk = base_blk + (1 if rem else 0)
    nsup = _ceil_div(max_blk + 1, 6)

    @functools.partial(
        pl.kernel,
        out_type=jax.ShapeDtypeStruct((NC, n_pad, ch), jnp.float32),
        mesh=_sc_mesh(),
        scratch_types=[
            pltpu.VMEM((3, CHUNK), jnp.int32),
            pltpu.VMEM((3, CHUNK), jnp.int32),
            pltpu.VMEM((CHUNK, ch), jnp.float32),
            pltpu.VMEM((CHUNK, ch), jnp.float32),
            pltpu.VMEM_SHARED((n_pad, ch), jnp.float32),
            pltpu.SemaphoreType.DMA,
            pltpu.SemaphoreType.DMA,
            pltpu.SemaphoreType.DMA,
            pltpu.SemaphoreType.DMA,
            pltpu.SemaphoreType.DMA,
            pltpu.SemaphoreType.DMA,
            pltpu.SemaphoreType.DMA,
        ],
        compiler_params=pltpu.CompilerParams(use_tc_tiling_on_sc=False),
    )
    def prop_kernel(u_hbm, src_hbm, dstm_hbm, part_out,
                    srcb, dstb, rows0, rows1, acc_sh,
                    si0, si1, si2, sg0, sg1, ss0, ss1):
        cid = lax.axis_index("c")
        sid = lax.axis_index("s")
        wid = _wid(cid, sid)
        nblk = jnp.where(wid < rem, base_blk + 1, base_blk)
        cb = base_blk * wid + jnp.minimum(wid, rem)
        ebase = cb * CHUNK

        rows = (rows0, rows1)
        sis = (si0, si1, si2)
        sgs = (sg0, sg1)
        sss = (ss0, ss1)

        def idx_start(c, slot):
            pltpu.async_copy(src_hbm.at[pl.ds(ebase + c * CHUNK, CHUNK)],
                             srcb.at[slot], sis[slot])
            pltpu.async_copy(dstm_hbm.at[pl.ds(cb + c, 1)],
                             dstb.at[pl.ds(slot, 1)], sis[slot])

        def idx_wait(slot):
            for _ in range(2):
                pltpu.make_async_copy(src_hbm.at[pl.ds(0, CHUNK)],
                                      srcb.at[slot], sis[slot]).wait()

        def rows_wait(b, sem):
            pltpu.make_async_copy(u_hbm.at[pl.ds(0, CHUNK)], rows[b], sem).wait()

        idx_start(0, 0)

        zeros16 = jnp.zeros((LANES,), jnp.float32)

        @pl.loop(0, CHUNK)
        def _fill(i):
            for j in range(ch // LANES):
                rows0[i, pl.ds(j * LANES, LANES)] = zeros16

        @pl.loop(0, zchunks)
        def _zero(i):
            pltpu.sync_copy(
                rows0, acc_sh.at[pl.ds(sid * rows_per_tile + i * CHUNK, CHUNK)]
            )

        plsc.subcore_barrier()

        @pl.loop(0, nsup)
        def _pipe(i):
            for t in range(6):
                j = i * 6 + t
                b = t % 2
                b2 = (t + 1) % 2
                slot = t % 3
                slot_n = (t + 1) % 3
                slot_p = (t + 2) % 3

                @pl.when(j < nblk)
                def _gather():
                    idx_wait(slot)

                    @pl.when(j >= 2)
                    def _wait_scatter():
                        rows_wait(b, sss[b])

                    half = CHUNK // 2
                    for h in range(2):
                        pltpu.async_copy(
                            u_hbm.at[srcb.at[slot, pl.ds(h * half, half)]],
                            rows[b].at[pl.ds(h * half, half)],
                            sgs[b],
                        )

                @pl.when(j + 1 < nblk)
                def _prefetch():
                    idx_start(j + 1, slot_n)

                jc = j - 1

                @pl.when((j >= 1) & (jc < nblk))
                def _scatter():
                    rows_wait(b2, sgs[b2])
                    pltpu.async_copy(rows[b2], acc_sh.at[dstb.at[slot_p]],
                                     sss[b2], add=True)

        for b in range(2):
            rows_wait(b, sss[b])

        plsc.subcore_barrier()
        off = pl.multiple_of(sid * rows_per_tile, 8)
        pltpu.sync_copy(
            acc_sh.at[pl.ds(off, rows_per_tile)],
            part_out.at[cid, pl.ds(off, rows_per_tile)],
        )

    return prop_kernel(u, src, dstm2)


_ROWCH = 50


def _scale_sc_call(part, dinv16, q16, t_prev, u_prev, n, n_pad, ch,
                   first, want_u):
    nrch = n // _ROWCH
    iters = _ceil_div(nrch, NW)
    a = -1.0 if first else -2.0

    n_in = 4 if first else 6
    out_type = [jax.ShapeDtypeStruct((n, ch), jnp.float32)]
    if want_u:
        out_type.append(jax.ShapeDtypeStruct((n, ch), jnp.float32))

    scratch = [
        pltpu.VMEM((_ROWCH, ch), jnp.float32),
        pltpu.VMEM((_ROWCH, ch), jnp.float32),
        pltpu.VMEM((_ROWCH, LANES), jnp.float32),
        pltpu.VMEM((_ROWCH, LANES), jnp.float32),
        pltpu.VMEM((_ROWCH, ch), jnp.float32),
        pltpu.VMEM((_ROWCH, ch), jnp.float32),
        pltpu.VMEM((_ROWCH, ch), jnp.float32),
        pltpu.VMEM((_ROWCH, ch), jnp.float32),
        pltpu.SemaphoreType.DMA,
    ]

    @functools.partial(
        pl.kernel,
        out_type=out_type,
        mesh=_sc_mesh(),
        scratch_types=scratch,
        compiler_params=pltpu.CompilerParams(use_tc_tiling_on_sc=False),
    )
    def scale_kernel(*refs):
        part_h, dinv_h, q_h = refs[0], refs[1], refs[2]
        idx = 3
        if not first:
            tp_h, up_h = refs[idx], refs[idx + 1]
            idx += 2
        t_out = refs[idx]
        idx += 1
        if want_u:
            u_out = refs[idx]
            idx += 1
        (p0_b, p1_b, dv_b, qv_b, tp_b, up_b, t_b, u_b, sem) = refs[idx:]

        cid = lax.axis_index("c")
        sid = lax.axis_index("s")
        wid = _wid(cid, sid)

        @pl.loop(0, iters)
        def _chunks(i):
            r = i * NW + wid

            @pl.when(r < nrch)
            def _():
                row = r * _ROWCH
                n_dma = 4 if first else 6
                pltpu.async_copy(part_h.at[0, pl.ds(row, _ROWCH)], p0_b, sem)
                pltpu.async_copy(part_h.at[1, pl.ds(row, _ROWCH)], p1_b, sem)
                pltpu.async_copy(dinv_h.at[pl.ds(row, _ROWCH)], dv_b, sem)
                pltpu.async_copy(q_h.at[pl.ds(row, _ROWCH)], qv_b, sem)
                if not first:
                    pltpu.async_copy(tp_h.at[pl.ds(row, _ROWCH)], tp_b, sem)
                    pltpu.async_copy(up_h.at[pl.ds(row, _ROWCH)], up_b, sem)
                for buf in (p0_b, p1_b, dv_b, qv_b, tp_b, up_b)[:n_dma]:
                    pltpu.make_async_copy(part_h.at[0, pl.ds(0, _ROWCH)]
                                          if buf.shape == (_ROWCH, ch)
                                          else dinv_h.at[pl.ds(0, _ROWCH)],
                                          buf, sem).wait()

                @pl.loop(0, _ROWCH)
                def _rows(rr):
                    dv = dv_b[rr, pl.ds(0, LANES)][0]
                    qv = qv_b[rr, pl.ds(0, LANES)][0]
                    for g in range(ch // LANES):
                        sl = pl.ds(g * LANES, LANES)
                        s = p0_b[rr, sl] + p1_b[rr, sl]
                        t = (a * dv) * s
                        if not first:
                            t = t - tp_b[rr, sl]
                        t_b[rr, sl] = t
                        if want_u:
                            u = (a * qv) * s
                            if not first:
                                u = u - up_b[rr, sl]
                            u_b[rr, sl] = u

                pltpu.sync_copy(t_b, t_out.at[pl.ds(row, _ROWCH)])
                if want_u:
                    pltpu.sync_copy(u_b, u_out.at[pl.ds(row, _ROWCH)])

    inputs = [part, dinv16, q16]
    if not first:
        inputs += [t_prev, u_prev]
    out = scale_kernel(*inputs)
    if want_u:
        return out[0], out[1]
    return out[0], None


def _round_up(v, m):
    return (v + m - 1) // m * m


def _ceil_div(a, b):
    return (a + b - 1) // b


_BN = 1000


def _prep_call(x, deg_part, n, ch):
    grid = n // _BN

    def body(deg_ref, x_ref, dinv_ref, q_ref, u0_ref):
        deg = deg_ref[0, :, :1] + deg_ref[1, :, :1]
        dinv = jnp.where(deg > 0.0, lax.rsqrt(jnp.maximum(deg, 1e-30)), 0.0)
        q = dinv * dinv
        dinv_ref[...] = jnp.broadcast_to(dinv, (_BN, LANES))
        q_ref[...] = jnp.broadcast_to(q, (_BN, LANES))
        u0_ref[...] = x_ref[...] * dinv

    return pl.pallas_call(
        body,
        grid=(grid,),
        in_specs=[
            pl.BlockSpec((NC, _BN, LANES), lambda i: (0, i, 0)),
            pl.BlockSpec((_BN, ch), lambda i: (i, 0)),
        ],
        out_specs=[
            pl.BlockSpec((_BN, LANES), lambda i: (i, 0)),
            pl.BlockSpec((_BN, LANES), lambda i: (i, 0)),
            pl.BlockSpec((_BN, ch), lambda i: (i, 0)),
        ],
        out_shape=[
            jax.ShapeDtypeStruct((n, LANES), jnp.float32),
            jax.ShapeDtypeStruct((n, LANES), jnp.float32),
            jax.ShapeDtypeStruct((n, ch), jnp.float32),
        ],
    )(deg_part, x)


def _scale_call(part, dinv16, q16, t_prev, u_prev, n, ch, first, want_u):
    grid = n // _BN
    a = -1.0 if first else -2.0

    def body(*refs):
        if first:
            part_ref, dinv_ref, q_ref = refs[:3]
            out_refs = refs[3:]
        else:
            part_ref, dinv_ref, q_ref, tp_ref, up_ref = refs[:5]
            out_refs = refs[5:]
        s = part_ref[0] + part_ref[1]
        dv = dinv_ref[:, :1]
        t = a * dv * s
        if not first:
            t = t - tp_ref[...]
        out_refs[0][...] = t
        if want_u:
            qv = q_ref[:, :1]
            u = a * qv * s
            if not first:
                u = u - up_ref[...]
            out_refs[1][...] = u

    in_specs = [
        pl.BlockSpec((NC, _BN, ch), lambda i: (0, i, 0)),
        pl.BlockSpec((_BN, LANES), lambda i: (i, 0)),
        pl.BlockSpec((_BN, LANES), lambda i: (i, 0)),
    ]
    inputs = [part, dinv16, q16]
    if not first:
        in_specs += [
            pl.BlockSpec((_BN, ch), lambda i: (i, 0)),
            pl.BlockSpec((_BN, ch), lambda i: (i, 0)),
        ]
        inputs += [t_prev, u_prev]
    n_out = 2 if want_u else 1
    out = pl.pallas_call(
        body,
        grid=(grid,),
        in_specs=in_specs,
        out_specs=[pl.BlockSpec((_BN, ch), lambda i: (i, 0))] * n_out,
        out_shape=[jax.ShapeDtypeStruct((n, ch), jnp.float32)] * n_out,
    )(*inputs)
    if want_u:
        return out[0], out[1]
    return out[0], None


def _matmul_call(ts, wbig, bias, n, ch, out_ch):
    grid = n // _BN
    nk = len(ts)

    def body(*refs):
        t_refs = refs[:nk]
        w_ref, b_ref = refs[nk], refs[nk + 1]
        acc = b_ref[...].astype(jnp.float32)
        acc = jnp.broadcast_to(acc, (_BN, out_ch))
        for k in range(nk):
            acc = acc + lax.dot_general(
                t_refs[k][...],
                w_ref[k],
                (((1,), (0,)), ((), ())),
                preferred_element_type=jnp.float32,
            )
        refs[nk + 2][...] = acc

    return pl.pallas_call(
        body,
        grid=(grid,),
        in_specs=[pl.BlockSpec((_BN, ch), lambda i: (i, 0)) for _ in range(nk)]
        + [
            pl.BlockSpec((nk, ch, out_ch), lambda i: (0, 0, 0)),
            pl.BlockSpec((1, out_ch), lambda i: (0, 0)),
        ],
        out_specs=pl.BlockSpec((_BN, out_ch), lambda i: (i, 0)),
        out_shape=jax.ShapeDtypeStruct((n, out_ch), jnp.float32),
    )(*ts, wbig, bias)


def kernel(x, edge_index, W4, b4, W5, b5, W6, b6, W7, b7):
    n, ch = x.shape
    e = edge_index.shape[1]
    ws = [W4, W5, W6, W7]
    bs = [b4, b5, b6, b7]
    kmax = max(w.shape[0] for w in ws)

    wbig = jnp.concatenate(
        [jnp.pad(w, ((0, kmax - w.shape[0]), (0, 0), (0, 0))) for w in ws], axis=2
    )
    bias = jnp.concatenate(bs).reshape(1, -1)
    out_ch = bias.shape[1]

    src = edge_index[0]
    dst = edge_index[1]

    deg_part, dstm = _degree_call(src, dst, n, e)
    dstm = dstm.reshape(e // CHUNK, CHUNK)
    dinv16, q16, u0 = _prep_call(x, deg_part, n, ch)

    t_list = [x]
    u_list = [u0]
    for k in range(1, kmax):
        part = _propagate_call(u_list[k - 1], src, dstm, n, e, ch)
        want_u = k < kmax - 1
        if k == 1:
            t_k, u_k = _scale_sc_call(part, dinv16, q16, None, None, n,
                                      0, ch, first=True, want_u=want_u)
        else:
            t_k, u_k = _scale_sc_call(part, dinv16, q16, t_list[k - 2],
                                      u_list[k - 2], n, 0, ch,
                                      first=False, want_u=want_u)
        t_list.append(t_k)
        u_list.append(u_k)

    return _matmul_call(t_list, wbig, bias, n, ch, out_ch)


# speedup vs baseline: 1.0021x; 1.0021x over previous
import functools

import jax
import jax.numpy as jnp
from jax import lax
from jax.experimental import pallas as pl
from jax.experimental.pallas import tpu as pltpu
from jax.experimental.pallas import tpu_sc as plsc

NC = 2
NS = 16
NW = NC * NS
LANES = 16
CHUNK = 128


def _sc_mesh():
    return plsc.VectorSubcoreMesh(
        core_axis_name="c", subcore_axis_name="s", num_cores=NC, num_subcores=NS
    )


def _wid(cid, sid):
    return sid * NC + cid


def _degree_call(src, dst, n, e):
    n_pad = _round_up(n + LANES, NS * CHUNK)
    rows_per_tile = n_pad // NS
    nchunk = e // CHUNK
    base_blk = nchunk // NW
    rem = nchunk % NW
    max_blk = base_blk + (1 if rem else 0)

    @functools.partial(
        pl.kernel,
        out_type=[
            jax.ShapeDtypeStruct((NC, n_pad, LANES), jnp.float32),
            jax.ShapeDtypeStruct((e,), jnp.int32),
        ],
        mesh=_sc_mesh(),
        scratch_types=[
            pltpu.VMEM((max_blk * CHUNK,), jnp.int32),
            pltpu.VMEM((max_blk * CHUNK,), jnp.int32),
            pltpu.VMEM((max_blk, CHUNK), jnp.int32),
            pltpu.VMEM((max_blk * CHUNK,), jnp.int32),
            pltpu.VMEM((CHUNK, LANES), jnp.float32),
            pltpu.VMEM((rows_per_tile, LANES), jnp.float32),
            pltpu.VMEM_SHARED((n_pad, LANES), jnp.float32),
            pltpu.SemaphoreType.DMA,
        ],
        compiler_params=pltpu.CompilerParams(use_tc_tiling_on_sc=False),
    )
    def deg_kernel(src_hbm, dst_hbm, deg_out, dstm_out,
                   srcs, dsts, srcm2, dstm_v, ones_v, zero_v, acc_sh, sem):
        cid = lax.axis_index("c")
        sid = lax.axis_index("s")
        wid = _wid(cid, sid)
        nblk = jnp.where(wid < rem, base_blk + 1, base_blk)
        cb = base_blk * wid + jnp.minimum(wid, rem)
        ebase = cb * CHUNK

        pltpu.sync_copy(src_hbm.at[pl.ds(ebase, base_blk * CHUNK)],
                        srcs.at[pl.ds(0, base_blk * CHUNK)])
        pltpu.sync_copy(dst_hbm.at[pl.ds(ebase, base_blk * CHUNK)],
                        dsts.at[pl.ds(0, base_blk * CHUNK)])
        if rem:
            @pl.when(wid < rem)
            def _tail():
                off = base_blk * CHUNK
                pltpu.sync_copy(src_hbm.at[pl.ds(ebase + off, CHUNK)],
                                srcs.at[pl.ds(off, CHUNK)])
                pltpu.sync_copy(dst_hbm.at[pl.ds(ebase + off, CHUNK)],
                                dsts.at[pl.ds(off, CHUNK)])

        zeros16 = jnp.zeros((LANES,), jnp.float32)
        e0 = jnp.where(lax.iota(jnp.int32, LANES) == 0, 1.0, 0.0)

        @pl.loop(0, CHUNK)
        def _fill(i):
            ones_v[i, :] = e0

        @pl.loop(0, rows_per_tile)
        def _fillz(i):
            zero_v[i, :] = zeros16

        pltpu.sync_copy(zero_v, acc_sh.at[pl.ds(sid * rows_per_tile,
                                                rows_per_tile)])

        dump = n + lax.iota(jnp.int32, LANES)

        @pl.loop(0, nblk)
        def _mask(r):
            for k in range(CHUNK // LANES):
                sl1 = pl.ds(r * CHUNK + k * LANES, LANES)
                s = srcs[sl1]
                d = dsts[sl1]
                is_loop = s == d
                srcm2[r, pl.ds(k * LANES, LANES)] = jnp.where(is_loop, dump, s)
                dstm_v[sl1] = jnp.where(is_loop, dump, d)

        pltpu.sync_copy(dstm_v.at[pl.ds(0, base_blk * CHUNK)],
                        dstm_out.at[pl.ds(ebase, base_blk * CHUNK)])
        if rem:
            @pl.when(wid < rem)
            def _tail2():
                off = base_blk * CHUNK
                pltpu.sync_copy(dstm_v.at[pl.ds(off, CHUNK)],
                                dstm_out.at[pl.ds(ebase + off, CHUNK)])

        plsc.subcore_barrier()

        @pl.loop(0, nblk)
        def _scat(r):
            pltpu.async_copy(ones_v, acc_sh.at[srcm2.at[r]], sem, add=True)

        @pl.loop(0, nblk)
        def _drain(r):
            pltpu.make_async_copy(
                deg_out.at[0, pl.ds(0, CHUNK)], ones_v, sem
            ).wait()

        plsc.subcore_barrier()
        off = pl.multiple_of(sid * rows_per_tile, 8)
        pltpu.sync_copy(
            acc_sh.at[pl.ds(off, rows_per_tile)],
            deg_out.at[cid, pl.ds(off, rows_per_tile)],
        )

    return deg_kernel(src, dst)


def _propagate_call(u, src, dstm2, n, e, ch):
    n_pad = _round_up(n + LANES, NS * CHUNK)
    rows_per_tile = n_pad // NS
    zchunks = rows_per_tile // CHUNK
    nchunk = e // CHUNK
    base_blk = nchunk // NW
    rem = nchunk % NW
    max_blk = base_blk + (1 if rem else 0)
    nsup = _ceil_div(max_blk + 1, 6)

    @functools.partial(
        pl.kernel,
        out_type=jax.ShapeDtypeStruct((NC, n_pad, ch), jnp.float32),
        mesh=_sc_mesh(),
        scratch_types=[
            pltpu.VMEM((3, CHUNK), jnp.int32),
            pltpu.VMEM((3, CHUNK), jnp.int32),
            pltpu.VMEM((CHUNK, ch), jnp.float32),
            pltpu.VMEM((CHUNK, ch), jnp.float32),
            pltpu.VMEM_SHARED((n_pad, ch), jnp.float32),
            pltpu.SemaphoreType.DMA,
            pltpu.SemaphoreType.DMA,
            pltpu.SemaphoreType.DMA,
            pltpu.SemaphoreType.DMA,
            pltpu.SemaphoreType.DMA,
            pltpu.SemaphoreType.DMA,
            pltpu.SemaphoreType.DMA,
        ],
        compiler_params=pltpu.CompilerParams(use_tc_tiling_on_sc=False),
    )
    def prop_kernel(u_hbm, src_hbm, dstm_hbm, part_out,
                    srcb, dstb, rows0, rows1, acc_sh,
                    si0, si1, si2, sg0, sg1, ss0, ss1):
        cid = lax.axis_index("c")
        sid = lax.axis_index("s")
        wid = _wid(cid, sid)
        nblk = jnp.where(wid < rem, base_blk + 1, base_blk)
        cb = base_blk * wid + jnp.minimum(wid, rem)
        ebase = cb * CHUNK

        rows = (rows0, rows1)
        sis = (si0, si1, si2)
        sgs = (sg0, sg1)
        sss = (ss0, ss1)

        def idx_start(c, slot):
            pltpu.async_copy(src_hbm.at[pl.ds(ebase + c * CHUNK, CHUNK)],
                             srcb.at[slot], sis[slot])
            pltpu.async_copy(dstm_hbm.at[pl.ds(cb + c, 1)],
                             dstb.at[pl.ds(slot, 1)], sis[slot])

        def idx_wait(slot):
            for _ in range(2):
                pltpu.make_async_copy(src_hbm.at[pl.ds(0, CHUNK)],
                                      srcb.at[slot], sis[slot]).wait()

        def rows_wait(b, sem):
            pltpu.make_async_copy(u_hbm.at[pl.ds(0, CHUNK)], rows[b], sem).wait()

        idx_start(0, 0)

        zeros16 = jnp.zeros((LANES,), jnp.float32)

        @pl.loop(0, CHUNK)
        def _fill(i):
            for j in range(ch // LANES):
                rows0[i, pl.ds(j * LANES, LANES)] = zeros16

        @pl.loop(0, zchunks)
        def _zero(i):
            pltpu.sync_copy(
                rows0, acc_sh.at[pl.ds(sid * rows_per_tile + i * CHUNK, CHUNK)]
            )

        plsc.subcore_barrier()

        @pl.loop(0, nsup)
        def _pipe(i):
            for t in range(6):
                j = i * 6 + t
                b = t % 2
                b2 = (t + 1) % 2
                slot = t % 3
                slot_n = (t + 1) % 3
                slot_p = (t + 2) % 3

                @pl.when(j < nblk)
                def _gather():
                    idx_wait(slot)

                    @pl.when(j >= 2)
                    def _wait_scatter():
                        rows_wait(b, sss[b])

                    pltpu.async_copy(u_hbm.at[srcb.at[slot]], rows[b], sgs[b])

                @pl.when(j + 1 < nblk)
                def _prefetch():
                    idx_start(j + 1, slot_n)

                jc = j - 1

                @pl.when((j >= 1) & (jc < nblk))
                def _scatter():
                    rows_wait(b2, sgs[b2])
                    pltpu.async_copy(rows[b2], acc_sh.at[dstb.at[slot_p]],
                                     sss[b2], add=True)

        for b in range(2):
            rows_wait(b, sss[b])

        plsc.subcore_barrier()
        off = pl.multiple_of(sid * rows_per_tile, 8)
        pltpu.sync_copy(
            acc_sh.at[pl.ds(off, rows_per_tile)],
            part_out.at[cid, pl.ds(off, rows_per_tile)],
        )

    return prop_kernel(u, src, dstm2)


_ROWCH = 100


def _scale_sc_call(part, dinv16, q16, t_prev, u_prev, n, n_pad, ch,
                   first, want_u):
    nrch = n // _ROWCH
    iters = _ceil_div(nrch, NW)
    a = -1.0 if first else -2.0

    n_in = 4 if first else 6
    out_type = [jax.ShapeDtypeStruct((n, ch), jnp.float32)]
    if want_u:
        out_type.append(jax.ShapeDtypeStruct((n, ch), jnp.float32))

    scratch = [
        pltpu.VMEM((_ROWCH, ch), jnp.float32),
        pltpu.VMEM((_ROWCH, ch), jnp.float32),
        pltpu.VMEM((_ROWCH, LANES), jnp.float32),
        pltpu.VMEM((_ROWCH, LANES), jnp.float32),
        pltpu.VMEM((_ROWCH, ch), jnp.float32),
        pltpu.VMEM((_ROWCH, ch), jnp.float32),
        pltpu.VMEM((_ROWCH, ch), jnp.float32),
        pltpu.VMEM((_ROWCH, ch), jnp.float32),
        pltpu.SemaphoreType.DMA,
    ]

    @functools.partial(
        pl.kernel,
        out_type=out_type,
        mesh=_sc_mesh(),
        scratch_types=scratch,
        compiler_params=pltpu.CompilerParams(use_tc_tiling_on_sc=False),
    )
    def scale_kernel(*refs):
        part_h, dinv_h, q_h = refs[0], refs[1], refs[2]
        idx = 3
        if not first:
            tp_h, up_h = refs[idx], refs[idx + 1]
            idx += 2
        t_out = refs[idx]
        idx += 1
        if want_u:
            u_out = refs[idx]
            idx += 1
        (p0_b, p1_b, dv_b, qv_b, tp_b, up_b, t_b, u_b, sem) = refs[idx:]

        cid = lax.axis_index("c")
        sid = lax.axis_index("s")
        wid = _wid(cid, sid)

        @pl.loop(0, iters)
        def _chunks(i):
            r = i * NW + wid

            @pl.when(r < nrch)
            def _():
                row = r * _ROWCH
                n_dma = 4 if first else 6
                pltpu.async_copy(part_h.at[0, pl.ds(row, _ROWCH)], p0_b, sem)
                pltpu.async_copy(part_h.at[1, pl.ds(row, _ROWCH)], p1_b, sem)
                pltpu.async_copy(dinv_h.at[pl.ds(row, _ROWCH)], dv_b, sem)
                pltpu.async_copy(q_h.at[pl.ds(row, _ROWCH)], qv_b, sem)
                if not first:
                    pltpu.async_copy(tp_h.at[pl.ds(row, _ROWCH)], tp_b, sem)
                    pltpu.async_copy(up_h.at[pl.ds(row, _ROWCH)], up_b, sem)
                for buf in (p0_b, p1_b, dv_b, qv_b, tp_b, up_b)[:n_dma]:
                    pltpu.make_async_copy(part_h.at[0, pl.ds(0, _ROWCH)]
                                          if buf.shape == (_ROWCH, ch)
                                          else dinv_h.at[pl.ds(0, _ROWCH)],
                                          buf, sem).wait()

                @pl.loop(0, _ROWCH)
                def _rows(rr):
                    dv = dv_b[rr, pl.ds(0, LANES)][0]
                    qv = qv_b[rr, pl.ds(0, LANES)][0]
                    for g in range(ch // LANES):
                        sl = pl.ds(g * LANES, LANES)
                        s = p0_b[rr, sl] + p1_b[rr, sl]
                        t = (a * dv) * s
                        if not first:
                            t = t - tp_b[rr, sl]
                        t_b[rr, sl] = t
                        if want_u:
                            u = (a * qv) * s
                            if not first:
                                u = u - up_b[rr, sl]
                            u_b[rr, sl] = u

                pltpu.sync_copy(t_b, t_out.at[pl.ds(row, _ROWCH)])
                if want_u:
                    pltpu.sync_copy(u_b, u_out.at[pl.ds(row, _ROWCH)])

    inputs = [part, dinv16, q16]
    if not first:
        inputs += [t_prev, u_prev]
    out = scale_kernel(*inputs)
    if want_u:
        return out[0], out[1]
    return out[0], None


def _round_up(v, m):
    return (v + m - 1) // m * m


def _ceil_div(a, b):
    return (a + b - 1) // b


_BN = 1000


def _prep_call(x, deg_part, n, ch):
    grid = n // _BN

    def body(deg_ref, x_ref, dinv_ref, q_ref, u0_ref):
        deg = deg_ref[0, :, :1] + deg_ref[1, :, :1]
        dinv = jnp.where(deg > 0.0, lax.rsqrt(jnp.maximum(deg, 1e-30)), 0.0)
        q = dinv * dinv
        dinv_ref[...] = jnp.broadcast_to(dinv, (_BN, LANES))
        q_ref[...] = jnp.broadcast_to(q, (_BN, LANES))
        u0_ref[...] = x_ref[...] * dinv

    return pl.pallas_call(
        body,
        grid=(grid,),
        in_specs=[
            pl.BlockSpec((NC, _BN, LANES), lambda i: (0, i, 0)),
            pl.BlockSpec((_BN, ch), lambda i: (i, 0)),
        ],
        out_specs=[
            pl.BlockSpec((_BN, LANES), lambda i: (i, 0)),
            pl.BlockSpec((_BN, LANES), lambda i: (i, 0)),
            pl.BlockSpec((_BN, ch), lambda i: (i, 0)),
        ],
        out_shape=[
            jax.ShapeDtypeStruct((n, LANES), jnp.float32),
            jax.ShapeDtypeStruct((n, LANES), jnp.float32),
            jax.ShapeDtypeStruct((n, ch), jnp.float32),
        ],
    )(deg_part, x)


def _scale_call(part, dinv16, q16, t_prev, u_prev, n, ch, first, want_u):
    grid = n // _BN
    a = -1.0 if first else -2.0

    def body(*refs):
        if first:
            part_ref, dinv_ref, q_ref = refs[:3]
            out_refs = refs[3:]
        else:
            part_ref, dinv_ref, q_ref, tp_ref, up_ref = refs[:5]
            out_refs = refs[5:]
        s = part_ref[0] + part_ref[1]
        dv = dinv_ref[:, :1]
        t = a * dv * s
        if not first:
            t = t - tp_ref[...]
        out_refs[0][...] = t
        if want_u:
            qv = q_ref[:, :1]
            u = a * qv * s
            if not first:
                u = u - up_ref[...]
            out_refs[1][...] = u

    in_specs = [
        pl.BlockSpec((NC, _BN, ch), lambda i: (0, i, 0)),
        pl.BlockSpec((_BN, LANES), lambda i: (i, 0)),
        pl.BlockSpec((_BN, LANES), lambda i: (i, 0)),
    ]
    inputs = [part, dinv16, q16]
    if not first:
        in_specs += [
            pl.BlockSpec((_BN, ch), lambda i: (i, 0)),
            pl.BlockSpec((_BN, ch), lambda i: (i, 0)),
        ]
        inputs += [t_prev, u_prev]
    n_out = 2 if want_u else 1
    out = pl.pallas_call(
        body,
        grid=(grid,),
        in_specs=in_specs,
        out_specs=[pl.BlockSpec((_BN, ch), lambda i: (i, 0))] * n_out,
        out_shape=[jax.ShapeDtypeStruct((n, ch), jnp.float32)] * n_out,
    )(*inputs)
    if want_u:
        return out[0], out[1]
    return out[0], None


def _matmul_call(ts, wbig, bias, n, ch, out_ch):
    grid = n // _BN
    nk = len(ts)

    def body(*refs):
        t_refs = refs[:nk]
        w_ref, b_ref = refs[nk], refs[nk + 1]
        acc = b_ref[...].astype(jnp.float32)
        acc = jnp.broadcast_to(acc, (_BN, out_ch))
        for k in range(nk):
            acc = acc + lax.dot_general(
                t_refs[k][...],
                w_ref[k],
                (((1,), (0,)), ((), ())),
                preferred_element_type=jnp.float32,
            )
        refs[nk + 2][...] = acc

    return pl.pallas_call(
        body,
        grid=(grid,),
        in_specs=[pl.BlockSpec((_BN, ch), lambda i: (i, 0)) for _ in range(nk)]
        + [
            pl.BlockSpec((nk, ch, out_ch), lambda i: (0, 0, 0)),
            pl.BlockSpec((1, out_ch), lambda i: (0, 0)),
        ],
        out_specs=pl.BlockSpec((_BN, out_ch), lambda i: (i, 0)),
        out_shape=jax.ShapeDtypeStruct((n, out_ch), jnp.float32),
    )(*ts, wbig, bias)


def kernel(x, edge_index, W4, b4, W5, b5, W6, b6, W7, b7):
    n, ch = x.shape
    e = edge_index.shape[1]
    ws = [W4, W5, W6, W7]
    bs = [b4, b5, b6, b7]
    kmax = max(w.shape[0] for w in ws)

    wbig = jnp.concatenate(
        [jnp.pad(w, ((0, kmax - w.shape[0]), (0, 0), (0, 0))) for w in ws], axis=2
    )
    bias = jnp.concatenate(bs).reshape(1, -1)
    out_ch = bias.shape[1]

    src = edge_index[0]
    dst = edge_index[1]

    deg_part, dstm = _degree_call(src, dst, n, e)
    dstm = dstm.reshape(e // CHUNK, CHUNK)
    dinv16, q16, u0 = _prep_call(x, deg_part, n, ch)

    t_list = [x]
    u_list = [u0]
    for k in range(1, kmax):
        part = _propagate_call(u_list[k - 1], src, dstm, n, e, ch)
        want_u = k < kmax - 1
        if k == 1:
            t_k, u_k = _scale_sc_call(part, dinv16, q16, None, None, n,
                                      0, ch, first=True, want_u=want_u)
        else:
            t_k, u_k = _scale_sc_call(part, dinv16, q16, t_list[k - 2],
                                      u_list[k - 2], n, 0, ch,
                                      first=False, want_u=want_u)
        t_list.append(t_k)
        u_list.append(u_k)

    return _matmul_call(t_list, wbig, bias, n, ch, out_ch)


# speedup vs baseline: 1.0408x; 1.0386x over previous
import functools

import jax
import jax.numpy as jnp
from jax import lax
from jax.experimental import pallas as pl
from jax.experimental.pallas import tpu as pltpu
from jax.experimental.pallas import tpu_sc as plsc

NC = 2
NS = 16
NW = NC * NS
LANES = 16
CHUNK = 128


def _sc_mesh():
    return plsc.VectorSubcoreMesh(
        core_axis_name="c", subcore_axis_name="s", num_cores=NC, num_subcores=NS
    )


def _wid(cid, sid):
    return sid * NC + cid


def _degree_call(src, dst, n, e):
    n_pad = _round_up(n + LANES, NS * CHUNK)
    rows_per_tile = n_pad // NS
    nchunk = e // CHUNK
    base_blk = nchunk // NW
    rem = nchunk % NW
    max_blk = base_blk + (1 if rem else 0)

    @functools.partial(
        pl.kernel,
        out_type=[
            jax.ShapeDtypeStruct((NC, n_pad, LANES), jnp.float32),
            jax.ShapeDtypeStruct((e,), jnp.int32),
        ],
        mesh=_sc_mesh(),
        scratch_types=[
            pltpu.VMEM((max_blk * CHUNK,), jnp.int32),
            pltpu.VMEM((max_blk * CHUNK,), jnp.int32),
            pltpu.VMEM((max_blk, CHUNK), jnp.int32),
            pltpu.VMEM((max_blk * CHUNK,), jnp.int32),
            pltpu.VMEM((CHUNK, LANES), jnp.float32),
            pltpu.VMEM((rows_per_tile, LANES), jnp.float32),
            pltpu.VMEM_SHARED((n_pad, LANES), jnp.float32),
            pltpu.SemaphoreType.DMA,
        ],
        compiler_params=pltpu.CompilerParams(use_tc_tiling_on_sc=False),
    )
    def deg_kernel(src_hbm, dst_hbm, deg_out, dstm_out,
                   srcs, dsts, srcm2, dstm_v, ones_v, zero_v, acc_sh, sem):
        cid = lax.axis_index("c")
        sid = lax.axis_index("s")
        wid = _wid(cid, sid)
        nblk = jnp.where(wid < rem, base_blk + 1, base_blk)
        cb = base_blk * wid + jnp.minimum(wid, rem)
        ebase = cb * CHUNK

        pltpu.sync_copy(src_hbm.at[pl.ds(ebase, base_blk * CHUNK)],
                        srcs.at[pl.ds(0, base_blk * CHUNK)])
        pltpu.sync_copy(dst_hbm.at[pl.ds(ebase, base_blk * CHUNK)],
                        dsts.at[pl.ds(0, base_blk * CHUNK)])
        if rem:
            @pl.when(wid < rem)
            def _tail():
                off = base_blk * CHUNK
                pltpu.sync_copy(src_hbm.at[pl.ds(ebase + off, CHUNK)],
                                srcs.at[pl.ds(off, CHUNK)])
                pltpu.sync_copy(dst_hbm.at[pl.ds(ebase + off, CHUNK)],
                                dsts.at[pl.ds(off, CHUNK)])

        zeros16 = jnp.zeros((LANES,), jnp.float32)
        e0 = jnp.where(lax.iota(jnp.int32, LANES) == 0, 1.0, 0.0)

        @pl.loop(0, CHUNK)
        def _fill(i):
            ones_v[i, :] = e0

        @pl.loop(0, rows_per_tile)
        def _fillz(i):
            zero_v[i, :] = zeros16

        pltpu.sync_copy(zero_v, acc_sh.at[pl.ds(sid * rows_per_tile,
                                                rows_per_tile)])

        dump = n + lax.iota(jnp.int32, LANES)

        @pl.loop(0, nblk)
        def _mask(r):
            for k in range(CHUNK // LANES):
                sl1 = pl.ds(r * CHUNK + k * LANES, LANES)
                s = srcs[sl1]
                d = dsts[sl1]
                is_loop = s == d
                srcm2[r, pl.ds(k * LANES, LANES)] = jnp.where(is_loop, dump, s)
                dstm_v[sl1] = jnp.where(is_loop, dump, d)

        pltpu.sync_copy(dstm_v.at[pl.ds(0, base_blk * CHUNK)],
                        dstm_out.at[pl.ds(ebase, base_blk * CHUNK)])
        if rem:
            @pl.when(wid < rem)
            def _tail2():
                off = base_blk * CHUNK
                pltpu.sync_copy(dstm_v.at[pl.ds(off, CHUNK)],
                                dstm_out.at[pl.ds(ebase + off, CHUNK)])

        plsc.subcore_barrier()

        @pl.loop(0, nblk)
        def _scat(r):
            pltpu.async_copy(ones_v, acc_sh.at[srcm2.at[r]], sem, add=True)

        @pl.loop(0, nblk)
        def _drain(r):
            pltpu.make_async_copy(
                deg_out.at[0, pl.ds(0, CHUNK)], ones_v, sem
            ).wait()

        plsc.subcore_barrier()
        off = pl.multiple_of(sid * rows_per_tile, 8)
        pltpu.sync_copy(
            acc_sh.at[pl.ds(off, rows_per_tile)],
            deg_out.at[cid, pl.ds(off, rows_per_tile)],
        )

    return deg_kernel(src, dst)


def _propagate_call(u, src, dstm2, n, e, ch):
    n_pad = _round_up(n + LANES, NS * CHUNK)
    rows_per_tile = n_pad // NS
    zchunks = rows_per_tile // CHUNK
    nchunk = e // CHUNK
    base_blk = nchunk // NW
    rem = nchunk % NW
    max_blk = base_blk + (1 if rem else 0)
    nsup = _ceil_div(max_blk + 1, 6)

    @functools.partial(
        pl.kernel,
        out_type=jax.ShapeDtypeStruct((NC, n_pad, ch), jnp.float32),
        mesh=_sc_mesh(),
        scratch_types=[
            pltpu.VMEM((3, CHUNK), jnp.int32),
            pltpu.VMEM((3, CHUNK), jnp.int32),
            pltpu.VMEM((CHUNK, ch), jnp.float32),
            pltpu.VMEM((CHUNK, ch), jnp.float32),
            pltpu.VMEM_SHARED((n_pad, ch), jnp.float32),
            pltpu.SemaphoreType.DMA,
            pltpu.SemaphoreType.DMA,
            pltpu.SemaphoreType.DMA,
            pltpu.SemaphoreType.DMA,
            pltpu.SemaphoreType.DMA,
            pltpu.SemaphoreType.DMA,
            pltpu.SemaphoreType.DMA,
        ],
        compiler_params=pltpu.CompilerParams(use_tc_tiling_on_sc=False),
    )
    def prop_kernel(u_hbm, src_hbm, dstm_hbm, part_out,
                    srcb, dstb, rows0, rows1, acc_sh,
                    si0, si1, si2, sg0, sg1, ss0, ss1):
        cid = lax.axis_index("c")
        sid = lax.axis_index("s")
        wid = _wid(cid, sid)
        nblk = jnp.where(wid < rem, base_blk + 1, base_blk)
        cb = base_blk * wid + jnp.minimum(wid, rem)
        ebase = cb * CHUNK

        rows = (rows0, rows1)
        sis = (si0, si1, si2)
        sgs = (sg0, sg1)
        sss = (ss0, ss1)

        def idx_start(c, slot):
            pltpu.async_copy(src_hbm.at[pl.ds(ebase + c * CHUNK, CHUNK)],
                             srcb.at[slot], sis[slot])
            pltpu.async_copy(dstm_hbm.at[pl.ds(cb + c, 1)],
                             dstb.at[pl.ds(slot, 1)], sis[slot])

        def idx_wait(slot):
            for _ in range(2):
                pltpu.make_async_copy(src_hbm.at[pl.ds(0, CHUNK)],
                                      srcb.at[slot], sis[slot]).wait()

        def rows_wait(b, sem):
            pltpu.make_async_copy(u_hbm.at[pl.ds(0, CHUNK)], rows[b], sem).wait()

        idx_start(0, 0)

        zeros16 = jnp.zeros((LANES,), jnp.float32)

        @pl.loop(0, CHUNK)
        def _fill(i):
            for j in range(ch // LANES):
                rows0[i, pl.ds(j * LANES, LANES)] = zeros16

        @pl.loop(0, zchunks)
        def _zero(i):
            pltpu.sync_copy(
                rows0, acc_sh.at[pl.ds(sid * rows_per_tile + i * CHUNK, CHUNK)]
            )

        plsc.subcore_barrier()

        @pl.loop(0, nsup)
        def _pipe(i):
            for t in range(6):
                j = i * 6 + t
                b = t % 2
                b2 = (t + 1) % 2
                slot = t % 3
                slot_n = (t + 1) % 3
                slot_p = (t + 2) % 3

                @pl.when(j < nblk)
                def _gather():
                    idx_wait(slot)

                    @pl.when(j >= 2)
                    def _wait_scatter():
                        rows_wait(b, sss[b])

                    pltpu.async_copy(u_hbm.at[srcb.at[slot]], rows[b], sgs[b])

                @pl.when(j + 1 < nblk)
                def _prefetch():
                    idx_start(j + 1, slot_n)

                jc = j - 1

                @pl.when((j >= 1) & (jc < nblk))
                def _scatter():
                    rows_wait(b2, sgs[b2])
                    pltpu.async_copy(rows[b2], acc_sh.at[dstb.at[slot_p]],
                                     sss[b2], add=True)

        for b in range(2):
            rows_wait(b, sss[b])

        plsc.subcore_barrier()
        off = pl.multiple_of(sid * rows_per_tile, 8)
        pltpu.sync_copy(
            acc_sh.at[pl.ds(off, rows_per_tile)],
            part_out.at[cid, pl.ds(off, rows_per_tile)],
        )

    return prop_kernel(u, src, dstm2)


_ROWCH = 100


def _scale_sc_call(part, dinv16, q16, t_prev, u_prev, n, n_pad, ch,
                   first, want_u):
    nrch = n // _ROWCH
    iters = _ceil_div(nrch, NW)
    a = -1.0 if first else -2.0

    n_in = 4 if first else 6
    out_type = [jax.ShapeDtypeStruct((n, ch), jnp.float32)]
    if want_u:
        out_type.append(jax.ShapeDtypeStruct((n, ch), jnp.float32))

    scratch = [
        pltpu.VMEM((_ROWCH, ch), jnp.float32),
        pltpu.VMEM((_ROWCH, ch), jnp.float32),
        pltpu.VMEM((_ROWCH, LANES), jnp.float32),
        pltpu.VMEM((_ROWCH, LANES), jnp.float32),
        pltpu.VMEM((_ROWCH, ch), jnp.float32),
        pltpu.VMEM((_ROWCH, ch), jnp.float32),
        pltpu.VMEM((_ROWCH, ch), jnp.float32),
        pltpu.VMEM((_ROWCH, ch), jnp.float32),
        pltpu.SemaphoreType.DMA,
    ]

    @functools.partial(
        pl.kernel,
        out_type=out_type,
        mesh=_sc_mesh(),
        scratch_types=scratch,
        compiler_params=pltpu.CompilerParams(use_tc_tiling_on_sc=False),
    )
    def scale_kernel(*refs):
        part_h, dinv_h, q_h = refs[0], refs[1], refs[2]
        idx = 3
        if not first:
            tp_h, up_h = refs[idx], refs[idx + 1]
            idx += 2
        t_out = refs[idx]
        idx += 1
        if want_u:
            u_out = refs[idx]
            idx += 1
        (p0_b, p1_b, dv_b, qv_b, tp_b, up_b, t_b, u_b, sem) = refs[idx:]

        cid = lax.axis_index("c")
        sid = lax.axis_index("s")
        wid = _wid(cid, sid)

        @pl.loop(0, iters)
        def _chunks(i):
            r = i * NW + wid

            @pl.when(r < nrch)
            def _():
                row = r * _ROWCH
                n_dma = 4 if first else 6
                pltpu.async_copy(part_h.at[0, pl.ds(row, _ROWCH)], p0_b, sem)
                pltpu.async_copy(part_h.at[1, pl.ds(row, _ROWCH)], p1_b, sem)
                pltpu.async_copy(dinv_h.at[pl.ds(row, _ROWCH)], dv_b, sem)
                pltpu.async_copy(q_h.at[pl.ds(row, _ROWCH)], qv_b, sem)
                if not first:
                    pltpu.async_copy(tp_h.at[pl.ds(row, _ROWCH)], tp_b, sem)
                    pltpu.async_copy(up_h.at[pl.ds(row, _ROWCH)], up_b, sem)
                for buf in (p0_b, p1_b, dv_b, qv_b, tp_b, up_b)[:n_dma]:
                    pltpu.make_async_copy(part_h.at[0, pl.ds(0, _ROWCH)]
                                          if buf.shape == (_ROWCH, ch)
                                          else dinv_h.at[pl.ds(0, _ROWCH)],
                                          buf, sem).wait()

                @pl.loop(0, _ROWCH)
                def _rows(rr):
                    dv = dv_b[rr, pl.ds(0, LANES)][0]
                    qv = qv_b[rr, pl.ds(0, LANES)][0]
                    for g in range(ch // LANES):
                        sl = pl.ds(g * LANES, LANES)
                        s = p0_b[rr, sl] + p1_b[rr, sl]
                        t = (a * dv) * s
                        if not first:
                            t = t - tp_b[rr, sl]
                        t_b[rr, sl] = t
                        if want_u:
                            u = (a * qv) * s
                            if not first:
                                u = u - up_b[rr, sl]
                            u_b[rr, sl] = u

                pltpu.sync_copy(t_b, t_out.at[pl.ds(row, _ROWCH)])
                if want_u:
                    pltpu.sync_copy(u_b, u_out.at[pl.ds(row, _ROWCH)])

    inputs = [part, dinv16, q16]
    if not first:
        inputs += [t_prev, u_prev]
    out = scale_kernel(*inputs)
    if want_u:
        return out[0], out[1]
    return out[0], None


def _round_up(v, m):
    return (v + m - 1) // m * m


def _ceil_div(a, b):
    return (a + b - 1) // b


_BN = 1000


def _prep_call(x, deg_part, n, ch):
    grid = n // _BN

    def body(deg_ref, x_ref, dinv_ref, q_ref, u0_ref):
        deg = deg_ref[0, :, :1] + deg_ref[1, :, :1]
        dinv = jnp.where(deg > 0.0, lax.rsqrt(jnp.maximum(deg, 1e-30)), 0.0)
        q = dinv * dinv
        dinv_ref[...] = jnp.broadcast_to(dinv, (_BN, LANES))
        q_ref[...] = jnp.broadcast_to(q, (_BN, LANES))
        u0_ref[...] = x_ref[...] * dinv

    return pl.pallas_call(
        body,
        grid=(grid,),
        in_specs=[
            pl.BlockSpec((NC, _BN, LANES), lambda i: (0, i, 0)),
            pl.BlockSpec((_BN, ch), lambda i: (i, 0)),
        ],
        out_specs=[
            pl.BlockSpec((_BN, LANES), lambda i: (i, 0)),
            pl.BlockSpec((_BN, LANES), lambda i: (i, 0)),
            pl.BlockSpec((_BN, ch), lambda i: (i, 0)),
        ],
        out_shape=[
            jax.ShapeDtypeStruct((n, LANES), jnp.float32),
            jax.ShapeDtypeStruct((n, LANES), jnp.float32),
            jax.ShapeDtypeStruct((n, ch), jnp.float32),
        ],
    )(deg_part, x)


def _scale_call(part, dinv16, q16, t_prev, u_prev, n, ch, first, want_u):
    grid = n // _BN
    a = -1.0 if first else -2.0

    def body(*refs):
        if first:
            part_ref, dinv_ref, q_ref = refs[:3]
            out_refs = refs[3:]
        else:
            part_ref, dinv_ref, q_ref, tp_ref, up_ref = refs[:5]
            out_refs = refs[5:]
        s = part_ref[0] + part_ref[1]
        dv = dinv_ref[:, :1]
        t = a * dv * s
        if not first:
            t = t - tp_ref[...]
        out_refs[0][...] = t
        if want_u:
            qv = q_ref[:, :1]
            u = a * qv * s
            if not first:
                u = u - up_ref[...]
            out_refs[1][...] = u

    in_specs = [
        pl.BlockSpec((NC, _BN, ch), lambda i: (0, i, 0)),
        pl.BlockSpec((_BN, LANES), lambda i: (i, 0)),
        pl.BlockSpec((_BN, LANES), lambda i: (i, 0)),
    ]
    inputs = [part, dinv16, q16]
    if not first:
        in_specs += [
            pl.BlockSpec((_BN, ch), lambda i: (i, 0)),
            pl.BlockSpec((_BN, ch), lambda i: (i, 0)),
        ]
        inputs += [t_prev, u_prev]
    n_out = 2 if want_u else 1
    out = pl.pallas_call(
        body,
        grid=(grid,),
        in_specs=in_specs,
        out_specs=[pl.BlockSpec((_BN, ch), lambda i: (i, 0))] * n_out,
        out_shape=[jax.ShapeDtypeStruct((n, ch), jnp.float32)] * n_out,
    )(*inputs)
    if want_u:
        return out[0], out[1]
    return out[0], None


def _matmul_call(ts, last_part, last_tprev_idx, dinv16, wbig, bias,
                 n, n_pad, ch, out_ch):
    grid = n // _BN
    nk = len(ts) + 1

    def body(*refs):
        t_refs = refs[: nk - 1]
        part_ref, dinv_ref, w_ref, b_ref = refs[nk - 1: nk + 3]
        dv = dinv_ref[:, :1]
        t_last = (-2.0 * dv) * (part_ref[0] + part_ref[1]) \
            - t_refs[last_tprev_idx][...]
        acc = b_ref[...].astype(jnp.float32)
        acc = jnp.broadcast_to(acc, (_BN, out_ch))
        for k in range(nk):
            t_blk = t_last if k == nk - 1 else t_refs[k][...]
            acc = acc + lax.dot_general(
                t_blk,
                w_ref[k],
                (((1,), (0,)), ((), ())),
                preferred_element_type=jnp.float32,
            )
        refs[nk + 3][...] = acc

    return pl.pallas_call(
        body,
        grid=(grid,),
        in_specs=[pl.BlockSpec((_BN, ch), lambda i: (i, 0))
                  for _ in range(nk - 1)]
        + [
            pl.BlockSpec((NC, _BN, ch), lambda i: (0, i, 0)),
            pl.BlockSpec((_BN, LANES), lambda i: (i, 0)),
            pl.BlockSpec((nk, ch, out_ch), lambda i: (0, 0, 0)),
            pl.BlockSpec((1, out_ch), lambda i: (0, 0)),
        ],
        out_specs=pl.BlockSpec((_BN, out_ch), lambda i: (i, 0)),
        out_shape=jax.ShapeDtypeStruct((n, out_ch), jnp.float32),
    )(*ts, last_part, dinv16, wbig, bias)


def kernel(x, edge_index, W4, b4, W5, b5, W6, b6, W7, b7):
    n, ch = x.shape
    e = edge_index.shape[1]
    ws = [W4, W5, W6, W7]
    bs = [b4, b5, b6, b7]
    kmax = max(w.shape[0] for w in ws)

    wbig = jnp.concatenate(
        [jnp.pad(w, ((0, kmax - w.shape[0]), (0, 0), (0, 0))) for w in ws], axis=2
    )
    bias = jnp.concatenate(bs).reshape(1, -1)
    out_ch = bias.shape[1]

    src = edge_index[0]
    dst = edge_index[1]

    deg_part, dstm = _degree_call(src, dst, n, e)
    dstm = dstm.reshape(e // CHUNK, CHUNK)
    dinv16, q16, u0 = _prep_call(x, deg_part, n, ch)

    t_list = [x]
    u_list = [u0]
    last_part = None
    for k in range(1, kmax):
        part = _propagate_call(u_list[k - 1], src, dstm, n, e, ch)
        if k == kmax - 1:
            last_part = part
            break
        want_u = k < kmax - 2
        if k == 1:
            t_k, u_k = _scale_sc_call(part, dinv16, q16, None, None, n,
                                      0, ch, first=True, want_u=True)
        else:
            t_k, u_k = _scale_sc_call(part, dinv16, q16, t_list[k - 2],
                                      u_list[k - 2], n, 0, ch,
                                      first=False, want_u=True)
        t_list.append(t_k)
        u_list.append(u_k)

    n_pad = _round_up(n + LANES, NS * CHUNK)
    return _matmul_call(t_list, last_part, kmax - 3, dinv16, wbig, bias,
                        n, n_pad, ch, out_ch)


# speedup vs baseline: 1.0937x; 1.0508x over previous
import functools

import jax
import jax.numpy as jnp
from jax import lax
from jax.experimental import pallas as pl
from jax.experimental.pallas import tpu as pltpu
from jax.experimental.pallas import tpu_sc as plsc

NC = 2
NS = 16
NW = NC * NS
LANES = 16
CHUNK = 128


def _sc_mesh():
    return plsc.VectorSubcoreMesh(
        core_axis_name="c", subcore_axis_name="s", num_cores=NC, num_subcores=NS
    )


def _wid(cid, sid):
    return sid * NC + cid


def _degree_call(src, dst, n, e):
    n_pad = _round_up(n + LANES, NS * CHUNK)
    rows_per_tile = n_pad // NS
    nchunk = e // CHUNK
    base_blk = nchunk // NW
    rem = nchunk % NW
    max_blk = base_blk + (1 if rem else 0)

    @functools.partial(
        pl.kernel,
        out_type=[
            jax.ShapeDtypeStruct((NC, n_pad, LANES), jnp.float32),
            jax.ShapeDtypeStruct((e,), jnp.int32),
        ],
        mesh=_sc_mesh(),
        scratch_types=[
            pltpu.VMEM((max_blk * CHUNK,), jnp.int32),
            pltpu.VMEM((max_blk * CHUNK,), jnp.int32),
            pltpu.VMEM((max_blk, CHUNK), jnp.int32),
            pltpu.VMEM((max_blk * CHUNK,), jnp.int32),
            pltpu.VMEM((CHUNK, LANES), jnp.float32),
            pltpu.VMEM((rows_per_tile, LANES), jnp.float32),
            pltpu.VMEM_SHARED((n_pad, LANES), jnp.float32),
            pltpu.SemaphoreType.DMA,
        ],
        compiler_params=pltpu.CompilerParams(use_tc_tiling_on_sc=False),
    )
    def deg_kernel(src_hbm, dst_hbm, deg_out, dstm_out,
                   srcs, dsts, srcm2, dstm_v, ones_v, zero_v, acc_sh, sem):
        cid = lax.axis_index("c")
        sid = lax.axis_index("s")
        wid = _wid(cid, sid)
        nblk = jnp.where(wid < rem, base_blk + 1, base_blk)
        cb = base_blk * wid + jnp.minimum(wid, rem)
        ebase = cb * CHUNK

        pltpu.sync_copy(src_hbm.at[pl.ds(ebase, base_blk * CHUNK)],
                        srcs.at[pl.ds(0, base_blk * CHUNK)])
        pltpu.sync_copy(dst_hbm.at[pl.ds(ebase, base_blk * CHUNK)],
                        dsts.at[pl.ds(0, base_blk * CHUNK)])
        if rem:
            @pl.when(wid < rem)
            def _tail():
                off = base_blk * CHUNK
                pltpu.sync_copy(src_hbm.at[pl.ds(ebase + off, CHUNK)],
                                srcs.at[pl.ds(off, CHUNK)])
                pltpu.sync_copy(dst_hbm.at[pl.ds(ebase + off, CHUNK)],
                                dsts.at[pl.ds(off, CHUNK)])

        zeros16 = jnp.zeros((LANES,), jnp.float32)
        e0 = jnp.where(lax.iota(jnp.int32, LANES) == 0, 1.0, 0.0)

        @pl.loop(0, CHUNK)
        def _fill(i):
            ones_v[i, :] = e0

        @pl.loop(0, rows_per_tile)
        def _fillz(i):
            zero_v[i, :] = zeros16

        pltpu.sync_copy(zero_v, acc_sh.at[pl.ds(sid * rows_per_tile,
                                                rows_per_tile)])

        dump = n + lax.iota(jnp.int32, LANES)

        @pl.loop(0, nblk)
        def _mask(r):
            for k in range(CHUNK // LANES):
                sl1 = pl.ds(r * CHUNK + k * LANES, LANES)
                s = srcs[sl1]
                d = dsts[sl1]
                is_loop = s == d
                srcm2[r, pl.ds(k * LANES, LANES)] = jnp.where(is_loop, dump, s)
                dstm_v[sl1] = jnp.where(is_loop, dump, d)

        pltpu.sync_copy(dstm_v.at[pl.ds(0, base_blk * CHUNK)],
                        dstm_out.at[pl.ds(ebase, base_blk * CHUNK)])
        if rem:
            @pl.when(wid < rem)
            def _tail2():
                off = base_blk * CHUNK
                pltpu.sync_copy(dstm_v.at[pl.ds(off, CHUNK)],
                                dstm_out.at[pl.ds(ebase + off, CHUNK)])

        plsc.subcore_barrier()

        @pl.loop(0, nblk)
        def _scat(r):
            pltpu.async_copy(ones_v, acc_sh.at[srcm2.at[r]], sem, add=True)

        @pl.loop(0, nblk)
        def _drain(r):
            pltpu.make_async_copy(
                deg_out.at[0, pl.ds(0, CHUNK)], ones_v, sem
            ).wait()

        plsc.subcore_barrier()
        off = pl.multiple_of(sid * rows_per_tile, 8)
        pltpu.sync_copy(
            acc_sh.at[pl.ds(off, rows_per_tile)],
            deg_out.at[cid, pl.ds(off, rows_per_tile)],
        )

    return deg_kernel(src, dst)


def _propagate_call(u, src, dstm2, n, e, ch):
    n_pad = _round_up(n + LANES, NS * CHUNK)
    rows_per_tile = n_pad // NS
    zchunks = rows_per_tile // CHUNK
    nchunk = e // CHUNK
    base_blk = nchunk // NW
    rem = nchunk % NW
    max_blk = base_blk + (1 if rem else 0)
    nsup = _ceil_div(max_blk + 1, 6)

    @functools.partial(
        pl.kernel,
        out_type=jax.ShapeDtypeStruct((NC, n_pad, ch), jnp.float32),
        mesh=_sc_mesh(),
        scratch_types=[
            pltpu.VMEM((3, CHUNK), jnp.int32),
            pltpu.VMEM((3, CHUNK), jnp.int32),
            pltpu.VMEM((CHUNK, ch), jnp.float32),
            pltpu.VMEM((CHUNK, ch), jnp.float32),
            pltpu.VMEM_SHARED((n_pad, ch), jnp.float32),
            pltpu.SemaphoreType.DMA,
            pltpu.SemaphoreType.DMA,
            pltpu.SemaphoreType.DMA,
            pltpu.SemaphoreType.DMA,
            pltpu.SemaphoreType.DMA,
            pltpu.SemaphoreType.DMA,
            pltpu.SemaphoreType.DMA,
        ],
        compiler_params=pltpu.CompilerParams(use_tc_tiling_on_sc=False),
    )
    def prop_kernel(u_hbm, src_hbm, dstm_hbm, part_out,
                    srcb, dstb, rows0, rows1, acc_sh,
                    si0, si1, si2, sg0, sg1, ss0, ss1):
        cid = lax.axis_index("c")
        sid = lax.axis_index("s")
        wid = _wid(cid, sid)
        nblk = jnp.where(wid < rem, base_blk + 1, base_blk)
        cb = base_blk * wid + jnp.minimum(wid, rem)
        ebase = cb * CHUNK

        rows = (rows0, rows1)
        sis = (si0, si1, si2)
        sgs = (sg0, sg1)
        sss = (ss0, ss1)

        def idx_start(c, slot):
            pltpu.async_copy(src_hbm.at[pl.ds(ebase + c * CHUNK, CHUNK)],
                             srcb.at[slot], sis[slot])
            pltpu.async_copy(dstm_hbm.at[pl.ds(cb + c, 1)],
                             dstb.at[pl.ds(slot, 1)], sis[slot])

        def idx_wait(slot):
            for _ in range(2):
                pltpu.make_async_copy(src_hbm.at[pl.ds(0, CHUNK)],
                                      srcb.at[slot], sis[slot]).wait()

        def rows_wait(b, sem):
            pltpu.make_async_copy(u_hbm.at[pl.ds(0, CHUNK)], rows[b], sem).wait()

        idx_start(0, 0)

        zeros16 = jnp.zeros((LANES,), jnp.float32)

        @pl.loop(0, CHUNK)
        def _fill(i):
            for j in range(ch // LANES):
                rows0[i, pl.ds(j * LANES, LANES)] = zeros16

        @pl.loop(0, zchunks)
        def _zero(i):
            pltpu.sync_copy(
                rows0, acc_sh.at[pl.ds(sid * rows_per_tile + i * CHUNK, CHUNK)]
            )

        plsc.subcore_barrier()

        @pl.loop(0, nsup)
        def _pipe(i):
            for t in range(6):
                j = i * 6 + t
                b = t % 2
                b2 = (t + 1) % 2
                slot = t % 3
                slot_n = (t + 1) % 3
                slot_p = (t + 2) % 3

                @pl.when(j < nblk)
                def _gather():
                    idx_wait(slot)

                    @pl.when(j >= 2)
                    def _wait_scatter():
                        rows_wait(b, sss[b])

                    pltpu.async_copy(u_hbm.at[srcb.at[slot]], rows[b], sgs[b])

                @pl.when(j + 1 < nblk)
                def _prefetch():
                    idx_start(j + 1, slot_n)

                jc = j - 1

                @pl.when((j >= 1) & (jc < nblk))
                def _scatter():
                    rows_wait(b2, sgs[b2])
                    pltpu.async_copy(rows[b2], acc_sh.at[dstb.at[slot_p]],
                                     sss[b2], add=True)

        for b in range(2):
            rows_wait(b, sss[b])

        plsc.subcore_barrier()
        off = pl.multiple_of(sid * rows_per_tile, 8)
        pltpu.sync_copy(
            acc_sh.at[pl.ds(off, rows_per_tile)],
            part_out.at[cid, pl.ds(off, rows_per_tile)],
        )

    return prop_kernel(u, src, dstm2)


_ROWCH = 50


def _scale_sc_call(part, dinv16, q16, t_prev, u_prev, n, n_pad, ch,
                   first, want_u):
    nrch = n // _ROWCH
    iters = _ceil_div(nrch, NW)
    a = -1.0 if first else -2.0
    n_dma = 4 if first else 6

    out_type = [jax.ShapeDtypeStruct((n, ch), jnp.float32)]
    if want_u:
        out_type.append(jax.ShapeDtypeStruct((n, ch), jnp.float32))

    bufset = [
        pltpu.VMEM((_ROWCH, ch), jnp.float32),
        pltpu.VMEM((_ROWCH, ch), jnp.float32),
        pltpu.VMEM((_ROWCH, LANES), jnp.float32),
        pltpu.VMEM((_ROWCH, LANES), jnp.float32),
        pltpu.VMEM((_ROWCH, ch), jnp.float32),
        pltpu.VMEM((_ROWCH, ch), jnp.float32),
    ]
    scratch = bufset + bufset + [
        pltpu.VMEM((_ROWCH, ch), jnp.float32),
        pltpu.VMEM((_ROWCH, ch), jnp.float32),
        pltpu.SemaphoreType.DMA,
        pltpu.SemaphoreType.DMA,
    ]

    @functools.partial(
        pl.kernel,
        out_type=out_type,
        mesh=_sc_mesh(),
        scratch_types=scratch,
        compiler_params=pltpu.CompilerParams(use_tc_tiling_on_sc=False),
    )
    def scale_kernel(*refs):
        part_h, dinv_h, q_h = refs[0], refs[1], refs[2]
        idx = 3
        if not first:
            tp_h, up_h = refs[idx], refs[idx + 1]
            idx += 2
        t_out = refs[idx]
        idx += 1
        if want_u:
            u_out = refs[idx]
            idx += 1
        bufs = (refs[idx:idx + 6], refs[idx + 6:idx + 12])
        t_b, u_b = refs[idx + 12], refs[idx + 13]
        sems = (refs[idx + 14], refs[idx + 15])

        cid = lax.axis_index("c")
        sid = lax.axis_index("s")
        wid = _wid(cid, sid)

        def fire(r, s):
            @pl.when(r < nrch)
            def _():
                row = r * _ROWCH
                p0_b, p1_b, dv_b, qv_b, tp_b, up_b = bufs[s]
                pltpu.async_copy(part_h.at[0, pl.ds(row, _ROWCH)], p0_b,
                                 sems[s])
                pltpu.async_copy(part_h.at[1, pl.ds(row, _ROWCH)], p1_b,
                                 sems[s])
                pltpu.async_copy(dinv_h.at[pl.ds(row, _ROWCH)], dv_b, sems[s])
                pltpu.async_copy(q_h.at[pl.ds(row, _ROWCH)], qv_b, sems[s])
                if not first:
                    pltpu.async_copy(tp_h.at[pl.ds(row, _ROWCH)], tp_b,
                                     sems[s])
                    pltpu.async_copy(up_h.at[pl.ds(row, _ROWCH)], up_b,
                                     sems[s])

        def drain(s):
            for buf in bufs[s][:n_dma]:
                pltpu.make_async_copy(part_h.at[0, pl.ds(0, _ROWCH)]
                                      if buf.shape == (_ROWCH, ch)
                                      else dinv_h.at[pl.ds(0, _ROWCH)],
                                      buf, sems[s]).wait()

        fire(wid, 0)

        @pl.loop(0, _ceil_div(iters, 2))
        def _chunks(i):
            for s in range(2):
                it = i * 2 + s
                r = it * NW + wid

                @pl.when(r < nrch)
                def _():
                    drain(s)
                    fire(r + NW, 1 - s)
                    p0_b, p1_b, dv_b, qv_b, tp_b, up_b = bufs[s]

                    @pl.loop(0, _ROWCH)
                    def _rows(rr):
                        dv = dv_b[rr, pl.ds(0, LANES)][0]
                        qv = qv_b[rr, pl.ds(0, LANES)][0]
                        for g in range(ch // LANES):
                            sl = pl.ds(g * LANES, LANES)
                            sv = p0_b[rr, sl] + p1_b[rr, sl]
                            t = (a * dv) * sv
                            if not first:
                                t = t - tp_b[rr, sl]
                            t_b[rr, sl] = t
                            if want_u:
                                u = (a * qv) * sv
                                if not first:
                                    u = u - up_b[rr, sl]
                                u_b[rr, sl] = u

                    row = r * _ROWCH
                    pltpu.sync_copy(t_b, t_out.at[pl.ds(row, _ROWCH)])
                    if want_u:
                        pltpu.sync_copy(u_b, u_out.at[pl.ds(row, _ROWCH)])

    inputs = [part, dinv16, q16]
    if not first:
        inputs += [t_prev, u_prev]
    out = scale_kernel(*inputs)
    if want_u:
        return out[0], out[1]
    return out[0], None


def _round_up(v, m):
    return (v + m - 1) // m * m


def _ceil_div(a, b):
    return (a + b - 1) // b


_BN = 1000


def _prep_call(x, deg_part, n, ch):
    grid = n // _BN

    def body(deg_ref, x_ref, dinv_ref, q_ref, u0_ref):
        deg = deg_ref[0, :, :1] + deg_ref[1, :, :1]
        dinv = jnp.where(deg > 0.0, lax.rsqrt(jnp.maximum(deg, 1e-30)), 0.0)
        q = dinv * dinv
        dinv_ref[...] = jnp.broadcast_to(dinv, (_BN, LANES))
        q_ref[...] = jnp.broadcast_to(q, (_BN, LANES))
        u0_ref[...] = x_ref[...] * dinv

    return pl.pallas_call(
        body,
        grid=(grid,),
        in_specs=[
            pl.BlockSpec((NC, _BN, LANES), lambda i: (0, i, 0)),
            pl.BlockSpec((_BN, ch), lambda i: (i, 0)),
        ],
        out_specs=[
            pl.BlockSpec((_BN, LANES), lambda i: (i, 0)),
            pl.BlockSpec((_BN, LANES), lambda i: (i, 0)),
            pl.BlockSpec((_BN, ch), lambda i: (i, 0)),
        ],
        out_shape=[
            jax.ShapeDtypeStruct((n, LANES), jnp.float32),
            jax.ShapeDtypeStruct((n, LANES), jnp.float32),
            jax.ShapeDtypeStruct((n, ch), jnp.float32),
        ],
    )(deg_part, x)


def _scale_call(part, dinv16, q16, t_prev, u_prev, n, ch, first, want_u):
    grid = n // _BN
    a = -1.0 if first else -2.0

    def body(*refs):
        if first:
            part_ref, dinv_ref, q_ref = refs[:3]
            out_refs = refs[3:]
        else:
            part_ref, dinv_ref, q_ref, tp_ref, up_ref = refs[:5]
            out_refs = refs[5:]
        s = part_ref[0] + part_ref[1]
        dv = dinv_ref[:, :1]
        t = a * dv * s
        if not first:
            t = t - tp_ref[...]
        out_refs[0][...] = t
        if want_u:
            qv = q_ref[:, :1]
            u = a * qv * s
            if not first:
                u = u - up_ref[...]
            out_refs[1][...] = u

    in_specs = [
        pl.BlockSpec((NC, _BN, ch), lambda i: (0, i, 0)),
        pl.BlockSpec((_BN, LANES), lambda i: (i, 0)),
        pl.BlockSpec((_BN, LANES), lambda i: (i, 0)),
    ]
    inputs = [part, dinv16, q16]
    if not first:
        in_specs += [
            pl.BlockSpec((_BN, ch), lambda i: (i, 0)),
            pl.BlockSpec((_BN, ch), lambda i: (i, 0)),
        ]
        inputs += [t_prev, u_prev]
    n_out = 2 if want_u else 1
    out = pl.pallas_call(
        body,
        grid=(grid,),
        in_specs=in_specs,
        out_specs=[pl.BlockSpec((_BN, ch), lambda i: (i, 0))] * n_out,
        out_shape=[jax.ShapeDtypeStruct((n, ch), jnp.float32)] * n_out,
    )(*inputs)
    if want_u:
        return out[0], out[1]
    return out[0], None


def _matmul_call(ts, last_part, last_tprev_idx, dinv16, wbig, bias,
                 n, n_pad, ch, out_ch):
    grid = n // _BN
    nk = len(ts) + 1

    def body(*refs):
        t_refs = refs[: nk - 1]
        part_ref, dinv_ref, w_ref, b_ref = refs[nk - 1: nk + 3]
        dv = dinv_ref[:, :1]
        t_last = (-2.0 * dv) * (part_ref[0] + part_ref[1]) \
            - t_refs[last_tprev_idx][...]
        acc = b_ref[...].astype(jnp.float32)
        acc = jnp.broadcast_to(acc, (_BN, out_ch))
        for k in range(nk):
            t_blk = t_last if k == nk - 1 else t_refs[k][...]
            acc = acc + lax.dot_general(
                t_blk,
                w_ref[k],
                (((1,), (0,)), ((), ())),
                preferred_element_type=jnp.float32,
            )
        refs[nk + 3][...] = acc

    return pl.pallas_call(
        body,
        grid=(grid,),
        in_specs=[pl.BlockSpec((_BN, ch), lambda i: (i, 0))
                  for _ in range(nk - 1)]
        + [
            pl.BlockSpec((NC, _BN, ch), lambda i: (0, i, 0)),
            pl.BlockSpec((_BN, LANES), lambda i: (i, 0)),
            pl.BlockSpec((nk, ch, out_ch), lambda i: (0, 0, 0)),
            pl.BlockSpec((1, out_ch), lambda i: (0, 0)),
        ],
        out_specs=pl.BlockSpec((_BN, out_ch), lambda i: (i, 0)),
        out_shape=jax.ShapeDtypeStruct((n, out_ch), jnp.float32),
    )(*ts, last_part, dinv16, wbig, bias)


def kernel(x, edge_index, W4, b4, W5, b5, W6, b6, W7, b7):
    n, ch = x.shape
    e = edge_index.shape[1]
    ws = [W4, W5, W6, W7]
    bs = [b4, b5, b6, b7]
    kmax = max(w.shape[0] for w in ws)

    wbig = jnp.concatenate(
        [jnp.pad(w, ((0, kmax - w.shape[0]), (0, 0), (0, 0))) for w in ws], axis=2
    )
    bias = jnp.concatenate(bs).reshape(1, -1)
    out_ch = bias.shape[1]

    src = edge_index[0]
    dst = edge_index[1]

    deg_part, dstm = _degree_call(src, dst, n, e)
    dstm = dstm.reshape(e // CHUNK, CHUNK)
    dinv16, q16, u0 = _prep_call(x, deg_part, n, ch)

    t_list = [x]
    u_list = [u0]
    last_part = None
    for k in range(1, kmax):
        part = _propagate_call(u_list[k - 1], src, dstm, n, e, ch)
        if k == kmax - 1:
            last_part = part
            break
        want_u = k < kmax - 2
        if k == 1:
            t_k, u_k = _scale_sc_call(part, dinv16, q16, None, None, n,
                                      0, ch, first=True, want_u=True)
        else:
            t_k, u_k = _scale_sc_call(part, dinv16, q16, t_list[k - 2],
                                      u_list[k - 2], n, 0, ch,
                                      first=False, want_u=True)
        t_list.append(t_k)
        u_list.append(u_k)

    n_pad = _round_up(n + LANES, NS * CHUNK)
    return _matmul_call(t_list, last_part, kmax - 3, dinv16, wbig, bias,
                        n, n_pad, ch, out_ch)
